# baseline (speedup 1.0000x reference)
import functools

import jax
import jax.numpy as jnp
from jax import lax
from jax.experimental import pallas as pl
from jax.experimental.pallas import tpu as pltpu
from jax.experimental.pallas import tpu_sc as plsc


_DIM = 32
_BATCH = 16384
_TPC = 16


@functools.lru_cache(maxsize=None)
def _build(V, D, B):
    info = plsc.get_sparse_core_info()
    NW = info.num_cores * info.num_subcores
    assert B % NW == 0
    b_per_w = B // NW
    n_chunks = b_per_w // _TPC
    mesh = plsc.VectorSubcoreMesh(core_axis_name="c", subcore_axis_name="s")

    @functools.partial(
        pl.kernel,
        mesh=mesh,
        out_type=jax.ShapeDtypeStruct((B, D), jnp.float32),
        scratch_types=[
            pltpu.VMEM((b_per_w,), jnp.int32),
            pltpu.VMEM((2, _TPC, 8, D), jnp.float32),
            pltpu.VMEM((b_per_w, D), jnp.float32),
            pltpu.SemaphoreType.DMA,
        ],
        compiler_params=pltpu.CompilerParams(disable_bounds_checks=True),
    )
    def gather_kernel(idx_hbm, table_hbm, out_hbm, t_v, tiles_v, rows_v, sem):
        wid = lax.axis_index("s") * info.num_cores + lax.axis_index("c")
        base = wid * b_per_w
        pltpu.sync_copy(idx_hbm.at[pl.ds(base, b_per_w)], t_v)

        def fire(c):
            tv = t_v[pl.ds(c * _TPC, _TPC)]
            buf = c % 2
            copies = []
            for j in range(_TPC):
                r8 = pl.multiple_of((tv[j] >> 3) << 3, 8)
                copies.append(
                    pltpu.async_copy(
                        table_hbm.at[pl.ds(r8, 8), :],
                        tiles_v.at[buf, j],
                        sem,
                    )
                )
            return copies

        pending = fire(0)
        for c in range(n_chunks):
            for cp in pending:
                cp.wait()
            nxt = fire(c + 1) if c + 1 < n_chunks else []
            buf = c % 2
            tv = t_v[pl.ds(c * _TPC, _TPC)]
            for j in range(_TPC):
                s = tv[j] & 7
                row = c * _TPC + j
                for h in range(D // 16):
                    rows_v[row, pl.ds(h * 16, 16)] = (
                        tiles_v[buf, j, s, pl.ds(h * 16, 16)]
                    )
            pending = nxt

        pltpu.sync_copy(rows_v, out_hbm.at[pl.ds(base, b_per_w)])

    return gather_kernel


def kernel(t, emb_weight):
    fn = _build(emb_weight.shape[0], _DIM, _BATCH)
    return fn(t.astype(jnp.int32), emb_weight)


# speedup vs baseline: 1.1347x; 1.1347x over previous
import functools

import jax
import jax.numpy as jnp
from jax import lax
from jax.experimental import pallas as pl
from jax.experimental.pallas import tpu as pltpu
from jax.experimental.pallas import tpu_sc as plsc


_DIM = 32
_BATCH = 16384
_NSEM = 8


@functools.lru_cache(maxsize=None)
def _build(V, D, B):
    info = plsc.get_sparse_core_info()
    NW = info.num_cores * info.num_subcores
    assert B % NW == 0
    b_per_w = B // NW
    mesh = plsc.VectorSubcoreMesh(core_axis_name="c", subcore_axis_name="s")

    @functools.partial(
        pl.kernel,
        mesh=mesh,
        out_type=jax.ShapeDtypeStruct((B, D), jnp.float32),
        scratch_types=[
            pltpu.VMEM((b_per_w,), jnp.int32),
            pltpu.VMEM((b_per_w, D), jnp.float32),
        ] + [pltpu.SemaphoreType.DMA] * _NSEM,
    )
    def gather_kernel(idx_hbm, table_hbm, out_hbm, t_v, rows_v, *sems):
        wid = lax.axis_index("s") * info.num_cores + lax.axis_index("c")
        base = wid * b_per_w
        pltpu.sync_copy(idx_hbm.at[pl.ds(base, b_per_w)], t_v)

        copies = []
        for k in range(b_per_w // 16):
            tv = t_v[pl.ds(k * 16, 16)]
            for j in range(16):
                i = k * 16 + j
                copies.append(
                    pltpu.async_copy(
                        table_hbm.at[pl.ds(tv[j], 1), :],
                        rows_v.at[pl.ds(i, 1), :],
                        sems[i % _NSEM],
                    )
                )
        for c in copies:
            c.wait()
        pltpu.sync_copy(rows_v, out_hbm.at[pl.ds(base, b_per_w)])

    return gather_kernel


def kernel(t, emb_weight):
    fn = _build(emb_weight.shape[0], _DIM, _BATCH)
    return fn(t.astype(jnp.int32), emb_weight)
